# baseline (device time: 175835 ns/iter reference)
import jax
import jax.numpy as jnp
from jax import lax
from jax.experimental import pallas as pl
from jax.experimental.pallas import tpu as pltpu

N_DEV = 4
M = 4096
N = 2048
NH = N // 2
CC = 2
CH = NH // CC
KS = 1024
MQ = M // N_DEV
MQ2 = MQ // 2
CW, CCW = 0, 1


def _body(xl_ref, w_ref, ycw_ref, yccw_ref, out_ref,
          stg_cw, stg_ccw,
          rcv_cw, rcv_ccw, sb_cw, sb_ccw, own_cw, own_ccw, amax_ref,
          rs_s_cw, rs_r_cw, rs_s_ccw, rs_r_ccw,
          ax_s, ax_r, ag_s_cw, ag_r_cw, ag_s_ccw, ag_r_ccw, st_sem):
    my = lax.axis_index("i")
    right = lax.rem(my + 1, N_DEV)
    left = lax.rem(my + N_DEV - 1, N_DEV)

    barrier = pltpu.get_barrier_semaphore()
    for nbr in (left, right):
        pl.semaphore_signal(barrier, inc=1, device_id=(nbr,),
                            device_id_type=pl.DeviceIdType.MESH)
    pl.semaphore_wait(barrier, 2)

    def wslice(ring, c):
        base = ring * NH + c * CH
        return w_ref[:, base:base + CH]

    def rs_rdma(s, ring, c):
        rcv = rcv_cw if ring == CW else rcv_ccw
        sb = sb_cw if ring == CW else sb_ccw
        return pltpu.make_async_remote_copy(
            src_ref=sb.at[c] if s == 0 else rcv.at[s - 1, c],
            dst_ref=rcv.at[s, c],
            send_sem=(rs_s_cw if ring == CW else rs_s_ccw).at[s, c],
            recv_sem=(rs_r_cw if ring == CW else rs_r_ccw).at[s, c],
            device_id=(right,) if ring == CW else (left,),
            device_id_type=pl.DeviceIdType.MESH,
        )

    cur = {}
    for c in range(CC):
        sb_cw[c] = jnp.dot(xl_ref[0], wslice(CW, c),
                           preferred_element_type=jnp.float32
                           ).astype(jnp.bfloat16)
        sb_ccw[c] = jnp.dot(xl_ref[0], wslice(CCW, c),
                            preferred_element_type=jnp.float32
                            ).astype(jnp.bfloat16)
        for ring in (CW, CCW):
            d = rs_rdma(0, ring, c)
            d.start()
            cur[(ring, c)] = d

    am_parts = []
    for s in range(N_DEV - 1):
        nxt = {}
        for c in range(CC):
            for ring in (CW, CCW):
                blk = s + 1 if ring == CW else 3 - s
                part = jnp.dot(xl_ref[blk], wslice(ring, c),
                               preferred_element_type=jnp.float32)
                cur[(ring, c)].wait()
                rcv = rcv_cw if ring == CW else rcv_ccw
                tot = rcv[s, c].astype(jnp.float32) + part
                if s < N_DEV - 2:
                    rcv[s, c] = tot.astype(jnp.bfloat16)
                    d = rs_rdma(s + 1, ring, c)
                    d.start()
                    nxt[(ring, c)] = d
                else:
                    own = own_cw if ring == CW else own_ccw
                    own[c] = tot
                    am_parts.append(jnp.max(tot))
        cur = nxt

    am = am_parts[0]
    for p in am_parts[1:]:
        am = jnp.maximum(am, p)
    am = jnp.maximum(am, 0.0)
    amax_ref[0, :, :] = jnp.full((8, 128), am, dtype=jnp.float32)
    opp = lax.rem(my + 2, N_DEV)
    ax_rdmas = []
    for j, (tgt, slot) in enumerate(((right, 3), (left, 1), (opp, 2))):
        rdma = pltpu.make_async_remote_copy(
            src_ref=amax_ref.at[0], dst_ref=amax_ref.at[slot],
            send_sem=ax_s.at[j], recv_sem=ax_r.at[slot - 1],
            device_id=(tgt,), device_id_type=pl.DeviceIdType.MESH)
        rdma.start()
        ax_rdmas.append(rdma)
    for rdma in ax_rdmas:
        rdma.wait()
    g_amax = jnp.max(amax_ref[:, :, :])
    scale = g_amax / 127.0
    inv_scale = 127.0 / g_amax

    for c in range(CC):
        qcw = jnp.clip(jnp.round(jnp.maximum(own_cw[c], 0.0) * inv_scale),
                       0.0, 127.0)
        qccw = jnp.clip(jnp.round(jnp.maximum(own_ccw[c], 0.0) * inv_scale),
                        0.0, 127.0)
        ycw_ref[0, :, c * CH:(c + 1) * CH] = qcw.astype(jnp.int8)
        yccw_ref[0, :, c * CH:(c + 1) * CH] = qccw.astype(jnp.int8)

    def emit(ring, k, prev):
        ybuf = ycw_ref if ring == CW else yccw_ref
        stg = stg_cw if ring == CW else stg_ccw
        q = lax.rem(my + 1 - k + N_DEV, N_DEV) if ring == CW \
            else lax.rem(my - 1 + k + N_DEV, N_DEV)
        cp = prev
        for h in range(2):
            if cp is not None:
                cp.wait()
            stg[:, :] = ybuf[k, h * MQ2:(h + 1) * MQ2, :
                             ].astype(jnp.float32) * scale
            cp = pltpu.make_async_copy(
                stg,
                out_ref.at[pl.ds(q * MQ + h * MQ2, MQ2),
                           ring * NH:(ring + 1) * NH],
                st_sem.at[ring])
            cp.start()
        return cp

    prev = {CW: None, CCW: None}
    for t in range(N_DEV - 1):
        cw = pltpu.make_async_remote_copy(
            src_ref=ycw_ref.at[t], dst_ref=ycw_ref.at[t + 1],
            send_sem=ag_s_cw.at[t], recv_sem=ag_r_cw.at[t],
            device_id=(right,), device_id_type=pl.DeviceIdType.MESH)
        ccw = pltpu.make_async_remote_copy(
            src_ref=yccw_ref.at[t], dst_ref=yccw_ref.at[t + 1],
            send_sem=ag_s_ccw.at[t], recv_sem=ag_r_ccw.at[t],
            device_id=(left,), device_id_type=pl.DeviceIdType.MESH)
        cw.start()
        ccw.start()
        prev[CW] = emit(CW, t, prev[CW])
        prev[CCW] = emit(CCW, t, prev[CCW])
        cw.wait()
        ccw.wait()
    prev[CW] = emit(CW, N_DEV - 1, prev[CW])
    prev[CCW] = emit(CCW, N_DEV - 1, prev[CCW])
    prev[CW].wait()
    prev[CCW].wait()


def kernel(x, w_mat):
    x16 = x.astype(jnp.bfloat16)
    w16 = w_mat.astype(jnp.bfloat16)
    my = lax.axis_index("i")

    sidx = (my - jnp.arange(N_DEV)) % N_DEV
    xl = x16.reshape(N_DEV, MQ, KS)[sidx]

    _ycw, _yccw, out = pl.pallas_call(
        _body,
        out_shape=[
            jax.ShapeDtypeStruct((N_DEV, MQ, NH), jnp.int8),
            jax.ShapeDtypeStruct((N_DEV, MQ, NH), jnp.int8),
            jax.ShapeDtypeStruct((M, N), jnp.float32),
        ],
        in_specs=[
            pl.BlockSpec(memory_space=pltpu.VMEM),
            pl.BlockSpec(memory_space=pltpu.VMEM),
        ],
        out_specs=[
            pl.BlockSpec(memory_space=pltpu.VMEM),
            pl.BlockSpec(memory_space=pltpu.VMEM),
            pl.BlockSpec(memory_space=pltpu.MemorySpace.HBM),
        ],
        scratch_shapes=[
            pltpu.VMEM((MQ2, NH), jnp.float32),
            pltpu.VMEM((MQ2, NH), jnp.float32),
            pltpu.VMEM((N_DEV - 1, CC, MQ, CH), jnp.bfloat16),
            pltpu.VMEM((N_DEV - 1, CC, MQ, CH), jnp.bfloat16),
            pltpu.VMEM((CC, MQ, CH), jnp.bfloat16),
            pltpu.VMEM((CC, MQ, CH), jnp.bfloat16),
            pltpu.VMEM((CC, MQ, CH), jnp.float32),
            pltpu.VMEM((CC, MQ, CH), jnp.float32),
            pltpu.VMEM((N_DEV, 8, 128), jnp.float32),
            pltpu.SemaphoreType.DMA((N_DEV - 1, CC)),
            pltpu.SemaphoreType.DMA((N_DEV - 1, CC)),
            pltpu.SemaphoreType.DMA((N_DEV - 1, CC)),
            pltpu.SemaphoreType.DMA((N_DEV - 1, CC)),
            pltpu.SemaphoreType.DMA((N_DEV - 1,)),
            pltpu.SemaphoreType.DMA((N_DEV - 1,)),
            pltpu.SemaphoreType.DMA((N_DEV - 1,)),
            pltpu.SemaphoreType.DMA((N_DEV - 1,)),
            pltpu.SemaphoreType.DMA((N_DEV - 1,)),
            pltpu.SemaphoreType.DMA((N_DEV - 1,)),
            pltpu.SemaphoreType.DMA((2,)),
        ],
        compiler_params=pltpu.CompilerParams(
            collective_id=0,
            vmem_limit_bytes=100 * 1024 * 1024,
        ),
    )(xl, w16)
    return out


# device time: 172076 ns/iter; 1.0218x vs baseline; 1.0218x over previous
import jax
import jax.numpy as jnp
from jax import lax
from jax.experimental import pallas as pl
from jax.experimental.pallas import tpu as pltpu

N_DEV = 4
M = 4096
N = 2048
NH = N // 2
CC = 2
CH = NH // CC
KS = 1024
MQ = M // N_DEV
MQ2 = MQ // 2
CW, CCW = 0, 1


def _body(xl_ref, w_ref, out_ref,
          ycw_ref, yccw_ref, stg_cw, stg_ccw,
          rcv_cw, rcv_ccw, sb_cw, sb_ccw, own_cw, own_ccw, amax_ref,
          rs_s_cw, rs_r_cw, rs_s_ccw, rs_r_ccw,
          ax_s, ax_r, ag_s_cw, ag_r_cw, ag_s_ccw, ag_r_ccw, st_sem):
    my = lax.axis_index("i")
    right = lax.rem(my + 1, N_DEV)
    left = lax.rem(my + N_DEV - 1, N_DEV)

    barrier = pltpu.get_barrier_semaphore()
    for nbr in (left, right):
        pl.semaphore_signal(barrier, inc=1, device_id=(nbr,),
                            device_id_type=pl.DeviceIdType.MESH)
    pl.semaphore_wait(barrier, 2)

    def wslice(ring, c):
        base = ring * NH + c * CH
        return w_ref[:, base:base + CH]

    def rs_rdma(s, ring, c):
        rcv = rcv_cw if ring == CW else rcv_ccw
        sb = sb_cw if ring == CW else sb_ccw
        return pltpu.make_async_remote_copy(
            src_ref=sb.at[c] if s == 0 else rcv.at[s - 1, c],
            dst_ref=rcv.at[s, c],
            send_sem=(rs_s_cw if ring == CW else rs_s_ccw).at[s, c],
            recv_sem=(rs_r_cw if ring == CW else rs_r_ccw).at[s, c],
            device_id=(right,) if ring == CW else (left,),
            device_id_type=pl.DeviceIdType.MESH,
        )

    cur = {}
    for c in range(CC):
        sb_cw[c] = jnp.dot(xl_ref[0], wslice(CW, c),
                           preferred_element_type=jnp.float32
                           ).astype(jnp.bfloat16)
        sb_ccw[c] = jnp.dot(xl_ref[0], wslice(CCW, c),
                            preferred_element_type=jnp.float32
                            ).astype(jnp.bfloat16)
        for ring in (CW, CCW):
            d = rs_rdma(0, ring, c)
            d.start()
            cur[(ring, c)] = d

    am_parts = []
    for s in range(N_DEV - 1):
        nxt = {}
        for c in range(CC):
            for ring in (CW, CCW):
                blk = s + 1 if ring == CW else 3 - s
                part = jnp.dot(xl_ref[blk], wslice(ring, c),
                               preferred_element_type=jnp.float32)
                cur[(ring, c)].wait()
                rcv = rcv_cw if ring == CW else rcv_ccw
                tot = rcv[s, c].astype(jnp.float32) + part
                if s < N_DEV - 2:
                    rcv[s, c] = tot.astype(jnp.bfloat16)
                    d = rs_rdma(s + 1, ring, c)
                    d.start()
                    nxt[(ring, c)] = d
                else:
                    own = own_cw if ring == CW else own_ccw
                    own[c] = tot
                    am_parts.append(jnp.max(tot))
        cur = nxt

    am = am_parts[0]
    for p in am_parts[1:]:
        am = jnp.maximum(am, p)
    am = jnp.maximum(am, 0.0)
    amax_ref[0, :, :] = jnp.full((8, 128), am, dtype=jnp.float32)
    opp = lax.rem(my + 2, N_DEV)
    ax_rdmas = []
    for j, (tgt, slot) in enumerate(((right, 3), (left, 1), (opp, 2))):
        rdma = pltpu.make_async_remote_copy(
            src_ref=amax_ref.at[0], dst_ref=amax_ref.at[slot],
            send_sem=ax_s.at[j], recv_sem=ax_r.at[slot - 1],
            device_id=(tgt,), device_id_type=pl.DeviceIdType.MESH)
        rdma.start()
        ax_rdmas.append(rdma)
    for rdma in ax_rdmas:
        rdma.wait()
    g_amax = jnp.max(amax_ref[:, :, :])
    scale = g_amax / 127.0
    inv_scale = 127.0 / g_amax

    for c in range(CC):
        qcw = jnp.clip(jnp.round(jnp.maximum(own_cw[c], 0.0) * inv_scale),
                       0.0, 127.0)
        qccw = jnp.clip(jnp.round(jnp.maximum(own_ccw[c], 0.0) * inv_scale),
                        0.0, 127.0)
        ycw_ref[0, :, c * CH:(c + 1) * CH] = qcw.astype(jnp.int8)
        yccw_ref[0, :, c * CH:(c + 1) * CH] = qccw.astype(jnp.int8)

    def emit(ring, k, prev):
        ybuf = ycw_ref if ring == CW else yccw_ref
        stg = stg_cw if ring == CW else stg_ccw
        q = lax.rem(my + 1 - k + N_DEV, N_DEV) if ring == CW \
            else lax.rem(my - 1 + k + N_DEV, N_DEV)
        cp = prev
        for h in range(2):
            if cp is not None:
                cp.wait()
            stg[:, :] = ybuf[k, h * MQ2:(h + 1) * MQ2, :
                             ].astype(jnp.float32) * scale
            cp = pltpu.make_async_copy(
                stg,
                out_ref.at[pl.ds(q * MQ + h * MQ2, MQ2),
                           ring * NH:(ring + 1) * NH],
                st_sem.at[ring])
            cp.start()
        return cp

    prev = {CW: None, CCW: None}
    for t in range(N_DEV - 1):
        cw = pltpu.make_async_remote_copy(
            src_ref=ycw_ref.at[t], dst_ref=ycw_ref.at[t + 1],
            send_sem=ag_s_cw.at[t], recv_sem=ag_r_cw.at[t],
            device_id=(right,), device_id_type=pl.DeviceIdType.MESH)
        ccw = pltpu.make_async_remote_copy(
            src_ref=yccw_ref.at[t], dst_ref=yccw_ref.at[t + 1],
            send_sem=ag_s_ccw.at[t], recv_sem=ag_r_ccw.at[t],
            device_id=(left,), device_id_type=pl.DeviceIdType.MESH)
        cw.start()
        ccw.start()
        prev[CW] = emit(CW, t, prev[CW])
        prev[CCW] = emit(CCW, t, prev[CCW])
        cw.wait()
        ccw.wait()
    prev[CW] = emit(CW, N_DEV - 1, prev[CW])
    prev[CCW] = emit(CCW, N_DEV - 1, prev[CCW])
    prev[CW].wait()
    prev[CCW].wait()


def kernel(x, w_mat):
    w16 = w_mat.astype(jnp.bfloat16)
    my = lax.axis_index("i")

    sidx = (my - jnp.arange(N_DEV)) % N_DEV
    xl = x.reshape(N_DEV, MQ, KS)[sidx].astype(jnp.bfloat16)

    out = pl.pallas_call(
        _body,
        out_shape=jax.ShapeDtypeStruct((M, N), jnp.float32),
        in_specs=[
            pl.BlockSpec(memory_space=pltpu.VMEM),
            pl.BlockSpec(memory_space=pltpu.VMEM),
        ],
        out_specs=pl.BlockSpec(memory_space=pltpu.MemorySpace.HBM),
        scratch_shapes=[
            pltpu.VMEM((N_DEV, MQ, NH), jnp.int8),
            pltpu.VMEM((N_DEV, MQ, NH), jnp.int8),
            pltpu.VMEM((MQ2, NH), jnp.float32),
            pltpu.VMEM((MQ2, NH), jnp.float32),
            pltpu.VMEM((N_DEV - 1, CC, MQ, CH), jnp.bfloat16),
            pltpu.VMEM((N_DEV - 1, CC, MQ, CH), jnp.bfloat16),
            pltpu.VMEM((CC, MQ, CH), jnp.bfloat16),
            pltpu.VMEM((CC, MQ, CH), jnp.bfloat16),
            pltpu.VMEM((CC, MQ, CH), jnp.float32),
            pltpu.VMEM((CC, MQ, CH), jnp.float32),
            pltpu.VMEM((N_DEV, 8, 128), jnp.float32),
            pltpu.SemaphoreType.DMA((N_DEV - 1, CC)),
            pltpu.SemaphoreType.DMA((N_DEV - 1, CC)),
            pltpu.SemaphoreType.DMA((N_DEV - 1, CC)),
            pltpu.SemaphoreType.DMA((N_DEV - 1, CC)),
            pltpu.SemaphoreType.DMA((N_DEV - 1,)),
            pltpu.SemaphoreType.DMA((N_DEV - 1,)),
            pltpu.SemaphoreType.DMA((N_DEV - 1,)),
            pltpu.SemaphoreType.DMA((N_DEV - 1,)),
            pltpu.SemaphoreType.DMA((N_DEV - 1,)),
            pltpu.SemaphoreType.DMA((N_DEV - 1,)),
            pltpu.SemaphoreType.DMA((2,)),
        ],
        compiler_params=pltpu.CompilerParams(
            collective_id=0,
            vmem_limit_bytes=100 * 1024 * 1024,
        ),
    )(xl, w16)
    return out


# device time: 172061 ns/iter; 1.0219x vs baseline; 1.0001x over previous
import jax
import jax.numpy as jnp
from jax import lax
from jax.experimental import pallas as pl
from jax.experimental.pallas import tpu as pltpu

N_DEV = 4
M = 4096
N = 2048
NH = N // 2
CC = 2
CH = NH // CC
KS = 1024
MQ = M // N_DEV
MQ2 = MQ // 2
CW, CCW = 0, 1


def _body(xl_ref, w_ref, out_ref,
          ycw_ref, yccw_ref, stg_cw, stg_ccw,
          rcv_cw, rcv_ccw, sb_cw, sb_ccw, own_cw, own_ccw, amax_ref,
          rs_s_cw, rs_r_cw, rs_s_ccw, rs_r_ccw,
          ax_s, ax_r, ag_s_cw, ag_r_cw, ag_s_ccw, ag_r_ccw, st_sem):
    my = lax.axis_index("i")
    right = lax.rem(my + 1, N_DEV)
    left = lax.rem(my + N_DEV - 1, N_DEV)

    barrier = pltpu.get_barrier_semaphore()
    for nbr in (left, right):
        pl.semaphore_signal(barrier, inc=1, device_id=(nbr,),
                            device_id_type=pl.DeviceIdType.MESH)
    pl.semaphore_wait(barrier, 2)

    def wslice(ring, c):
        base = ring * NH + c * CH
        return w_ref[:, base:base + CH]

    def rs_rdma(s, ring, c):
        rcv = rcv_cw if ring == CW else rcv_ccw
        sb = sb_cw if ring == CW else sb_ccw
        return pltpu.make_async_remote_copy(
            src_ref=sb.at[c] if s == 0 else rcv.at[s - 1, c],
            dst_ref=rcv.at[s, c],
            send_sem=(rs_s_cw if ring == CW else rs_s_ccw).at[s, c],
            recv_sem=(rs_r_cw if ring == CW else rs_r_ccw).at[s, c],
            device_id=(right,) if ring == CW else (left,),
            device_id_type=pl.DeviceIdType.MESH,
        )

    cur = {}
    for c in range(CC):
        sb_cw[c] = jnp.dot(xl_ref[0], wslice(CW, c),
                           preferred_element_type=jnp.float32
                           ).astype(jnp.bfloat16)
        sb_ccw[c] = jnp.dot(xl_ref[0], wslice(CCW, c),
                            preferred_element_type=jnp.float32
                            ).astype(jnp.bfloat16)
        for ring in (CW, CCW):
            d = rs_rdma(0, ring, c)
            d.start()
            cur[(ring, c)] = d

    am_parts = []
    for s in range(N_DEV - 1):
        nxt = {}
        for c in range(CC):
            for ring in (CW, CCW):
                blk = s + 1 if ring == CW else 3 - s
                part = jnp.dot(xl_ref[blk], wslice(ring, c),
                               preferred_element_type=jnp.float32)
                cur[(ring, c)].wait()
                rcv = rcv_cw if ring == CW else rcv_ccw
                tot = rcv[s, c].astype(jnp.float32) + part
                if s < N_DEV - 2:
                    rcv[s, c] = tot.astype(jnp.bfloat16)
                    d = rs_rdma(s + 1, ring, c)
                    d.start()
                    nxt[(ring, c)] = d
                else:
                    own = own_cw if ring == CW else own_ccw
                    own[c] = tot
                    am_parts.append(jnp.max(tot))
        cur = nxt

    am = am_parts[0]
    for p in am_parts[1:]:
        am = jnp.maximum(am, p)
    am = jnp.maximum(am, 0.0)
    amax_ref[0, :, :] = jnp.full((8, 128), am, dtype=jnp.float32)
    opp = lax.rem(my + 2, N_DEV)
    ax_rdmas = []
    for j, (tgt, slot) in enumerate(((right, 3), (left, 1), (opp, 2))):
        rdma = pltpu.make_async_remote_copy(
            src_ref=amax_ref.at[0], dst_ref=amax_ref.at[slot],
            send_sem=ax_s.at[j], recv_sem=ax_r.at[slot - 1],
            device_id=(tgt,), device_id_type=pl.DeviceIdType.MESH)
        rdma.start()
        ax_rdmas.append(rdma)
    for rdma in ax_rdmas:
        rdma.wait()
    g_amax = jnp.max(amax_ref[:, :, :])
    scale = g_amax / 127.0
    inv_scale = 127.0 / g_amax

    for c in range(CC):
        qcw = jnp.clip(jnp.round(jnp.maximum(own_cw[c], 0.0) * inv_scale),
                       0.0, 127.0)
        qccw = jnp.clip(jnp.round(jnp.maximum(own_ccw[c], 0.0) * inv_scale),
                        0.0, 127.0)
        ycw_ref[0, :, c * CH:(c + 1) * CH] = qcw.astype(jnp.int8)
        yccw_ref[0, :, c * CH:(c + 1) * CH] = qccw.astype(jnp.int8)

    def emit(ring, k, prev):
        ybuf = ycw_ref if ring == CW else yccw_ref
        stg = stg_cw if ring == CW else stg_ccw
        q = lax.rem(my + 1 - k + N_DEV, N_DEV) if ring == CW \
            else lax.rem(my - 1 + k + N_DEV, N_DEV)
        cp = prev
        for h in range(2):
            if cp is not None:
                cp.wait()
            stg[:, :] = ybuf[k, h * MQ2:(h + 1) * MQ2, :
                             ].astype(jnp.float32) * scale
            cp = pltpu.make_async_copy(
                stg,
                out_ref.at[pl.ds(q * MQ + h * MQ2, MQ2),
                           ring * NH:(ring + 1) * NH],
                st_sem.at[ring])
            cp.start()
        return cp

    prev = {CW: None, CCW: None}
    for t in range(N_DEV - 1):
        cw = pltpu.make_async_remote_copy(
            src_ref=ycw_ref.at[t], dst_ref=ycw_ref.at[t + 1],
            send_sem=ag_s_cw.at[t], recv_sem=ag_r_cw.at[t],
            device_id=(right,), device_id_type=pl.DeviceIdType.MESH)
        ccw = pltpu.make_async_remote_copy(
            src_ref=yccw_ref.at[t], dst_ref=yccw_ref.at[t + 1],
            send_sem=ag_s_ccw.at[t], recv_sem=ag_r_ccw.at[t],
            device_id=(left,), device_id_type=pl.DeviceIdType.MESH)
        cw.start()
        ccw.start()
        prev[CW] = emit(CW, t, prev[CW])
        prev[CCW] = emit(CCW, t, prev[CCW])
        cw.wait()
        ccw.wait()
    prev[CW] = emit(CW, N_DEV - 1, prev[CW])
    prev[CCW] = emit(CCW, N_DEV - 1, prev[CCW])
    prev[CW].wait()
    prev[CCW].wait()


def kernel(x, w_mat):
    w16 = w_mat.astype(jnp.bfloat16)
    my = lax.axis_index("i")

    sidx = (my - jnp.arange(N_DEV)) % N_DEV
    xl = x.reshape(N_DEV, MQ, KS)[sidx].astype(jnp.bfloat16)

    out = pl.pallas_call(
        _body,
        out_shape=jax.ShapeDtypeStruct((M, N), jnp.float32),
        in_specs=[
            pl.BlockSpec(memory_space=pltpu.VMEM),
            pl.BlockSpec(memory_space=pltpu.VMEM),
        ],
        out_specs=pl.BlockSpec(memory_space=pl.ANY),
        scratch_shapes=[
            pltpu.VMEM((N_DEV, MQ, NH), jnp.int8),
            pltpu.VMEM((N_DEV, MQ, NH), jnp.int8),
            pltpu.VMEM((MQ2, NH), jnp.float32),
            pltpu.VMEM((MQ2, NH), jnp.float32),
            pltpu.VMEM((N_DEV - 1, CC, MQ, CH), jnp.bfloat16),
            pltpu.VMEM((N_DEV - 1, CC, MQ, CH), jnp.bfloat16),
            pltpu.VMEM((CC, MQ, CH), jnp.bfloat16),
            pltpu.VMEM((CC, MQ, CH), jnp.bfloat16),
            pltpu.VMEM((CC, MQ, CH), jnp.float32),
            pltpu.VMEM((CC, MQ, CH), jnp.float32),
            pltpu.VMEM((N_DEV, 8, 128), jnp.float32),
            pltpu.SemaphoreType.DMA((N_DEV - 1, CC)),
            pltpu.SemaphoreType.DMA((N_DEV - 1, CC)),
            pltpu.SemaphoreType.DMA((N_DEV - 1, CC)),
            pltpu.SemaphoreType.DMA((N_DEV - 1, CC)),
            pltpu.SemaphoreType.DMA((N_DEV - 1,)),
            pltpu.SemaphoreType.DMA((N_DEV - 1,)),
            pltpu.SemaphoreType.DMA((N_DEV - 1,)),
            pltpu.SemaphoreType.DMA((N_DEV - 1,)),
            pltpu.SemaphoreType.DMA((N_DEV - 1,)),
            pltpu.SemaphoreType.DMA((N_DEV - 1,)),
            pltpu.SemaphoreType.DMA((2,)),
        ],
        compiler_params=pltpu.CompilerParams(
            collective_id=0,
            vmem_limit_bytes=100 * 1024 * 1024,
        ),
    )(xl, w16)
    return out
